# baseline (speedup 1.0000x reference)
import functools

import jax
import jax.numpy as jnp
import numpy as np
from jax import lax
from jax.experimental import pallas as pl
from jax.experimental.pallas import tpu as pltpu
from jax.experimental.pallas import tpu_sc as plsc

BATCH = 4096
SEQ = 200
SEQ_PAD = 208
EMBED_DIM = 32
NUM_CLASSES = 100
CLASS_PAD = 128

NUM_CORES = 2
NUM_SUBCORES = 16
NUM_WORKERS = NUM_CORES * NUM_SUBCORES
BPW = BATCH // NUM_WORKERS

_N16 = SEQ_PAD // 16
_G1 = 128
_G2 = SEQ_PAD - _G1
VOCAB_ROWS = 1000000
HALF_D = EMBED_DIM // 2

_PERM = np.concatenate([np.arange(0, EMBED_DIM, 2),
                        np.arange(1, EMBED_DIM, 2)])


def _sc_pool_sums(x_pad, table):
    mesh = plsc.VectorSubcoreMesh(core_axis_name="c", subcore_axis_name="s")

    @functools.partial(
        pl.kernel,
        mesh=mesh,
        compiler_params=pltpu.CompilerParams(use_tc_tiling_on_sc=False),
        out_type=jax.ShapeDtypeStruct((BATCH, EMBED_DIM), jnp.float32),
        scratch_types=[
            pltpu.VMEM((BPW, SEQ_PAD), jnp.int32),
            pltpu.VMEM((SEQ_PAD, HALF_D), jnp.int32),
            pltpu.VMEM((BPW, EMBED_DIM), jnp.float32),
            pltpu.SemaphoreType.DMA,
        ],
    )
    def body(x_hbm, table_hbm, out_hbm, idx_v, rows_v, sums_v, sem):
        wid = lax.axis_index("s") * NUM_CORES + lax.axis_index("c")
        base = wid * BPW
        pltpu.sync_copy(x_hbm.at[pl.ds(base, BPW)], idx_v)

        zero = jnp.zeros((16,), jnp.float32)

        def row_body(row, carry):
            cp1 = pltpu.async_copy(
                table_hbm.at[idx_v.at[row, pl.ds(0, _G1)]],
                rows_v.at[pl.ds(0, _G1)], sem)
            cp2 = pltpu.async_copy(
                table_hbm.at[idx_v.at[row, pl.ds(_G1, _G2)]],
                rows_v.at[pl.ds(_G1, _G2)], sem)
            cp1.wait()
            cp2.wait()

            def chunk(c, accs):
                accs = list(accs)
                r0 = c * 16
                himask = jnp.full((16,), -65536, jnp.int32)
                for u in range(16):
                    vi = rows_v[r0 + u, pl.ds(0, HALF_D)]
                    ev = lax.bitcast_convert_type(vi << 16, jnp.float32)
                    od = lax.bitcast_convert_type(vi & himask, jnp.float32)
                    p = (u % 4) * 2
                    accs[p] = accs[p] + ev
                    accs[p + 1] = accs[p + 1] + od
                return tuple(accs)

            accs = lax.fori_loop(0, _N16, chunk, (zero,) * 8)

            s0 = (accs[0] + accs[2]) + (accs[4] + accs[6])
            s1 = (accs[1] + accs[3]) + (accs[5] + accs[7])
            sums_v[row, pl.ds(0, 16)] = s0
            sums_v[row, pl.ds(16, 16)] = s1
            return carry

        lax.fori_loop(0, BPW, row_body, 0)
        pltpu.sync_copy(sums_v, out_hbm.at[pl.ds(base, BPW)])

    return body(x_pad, table)


def _tc_matmul(sums, x_pad, t0, w_scaled, b_pad):
    def body(s_ref, x_ref, t0_ref, w_ref, b_ref, o_ref):
        cnt0 = jnp.sum((x_ref[...] == 0).astype(jnp.float32), axis=1,
                       keepdims=True)
        pooled = s_ref[...] - cnt0 * t0_ref[...]
        o_ref[...] = jnp.dot(
            pooled, w_ref[...], preferred_element_type=jnp.float32
        ) + b_ref[...]

    blk = 1024
    return pl.pallas_call(
        body,
        grid=(BATCH // blk,),
        in_specs=[
            pl.BlockSpec((blk, EMBED_DIM), lambda i: (i, 0)),
            pl.BlockSpec((blk, SEQ_PAD), lambda i: (i, 0)),
            pl.BlockSpec((1, EMBED_DIM), lambda i: (0, 0)),
            pl.BlockSpec((EMBED_DIM, CLASS_PAD), lambda i: (0, 0)),
            pl.BlockSpec((1, CLASS_PAD), lambda i: (0, 0)),
        ],
        out_specs=pl.BlockSpec((blk, CLASS_PAD), lambda i: (i, 0)),
        out_shape=jax.ShapeDtypeStruct((BATCH, CLASS_PAD), jnp.float32),
    )(sums, x_pad, t0, w_scaled, b_pad)


def kernel(x, table, W, b):
    x_pad = jnp.pad(x, ((0, 0), (0, SEQ_PAD - SEQ)))
    ti = lax.bitcast_convert_type(table, jnp.int32)
    ev_bits = lax.slice(ti, (0, 0), (VOCAB_ROWS, EMBED_DIM), (1, 2))
    od_bits = lax.slice(ti, (0, 1), (VOCAB_ROWS, EMBED_DIM), (1, 2))
    rnd = jnp.int32(0x8000)
    lo = lax.shift_right_logical(ev_bits + rnd, 16)
    hi = (od_bits + rnd) & jnp.int32(-65536)
    table_packed = lo | hi

    t0_bf = lax.slice(table, (0, 0), (1, EMBED_DIM)).astype(
        jnp.bfloat16).astype(jnp.float32)
    t0p = t0_bf[:, _PERM]
    w_scaled = jnp.zeros((EMBED_DIM, CLASS_PAD), jnp.float32)
    w_scaled = w_scaled.at[:, :NUM_CLASSES].set((W.T * (1.0 / SEQ))[_PERM])
    b_pad = jnp.zeros((1, CLASS_PAD), jnp.float32).at[0, :NUM_CLASSES].set(b)

    sums = _sc_pool_sums(x_pad, table_packed)
    logits_pad = _tc_matmul(sums, x_pad, t0p, w_scaled, b_pad)
    return logits_pad[:, :NUM_CLASSES]


# speedup vs baseline: 9.1551x; 9.1551x over previous
import functools

import jax
import jax.numpy as jnp
from jax import lax
from jax.experimental import pallas as pl
from jax.experimental.pallas import tpu as pltpu
from jax.experimental.pallas import tpu_sc as plsc

BATCH = 4096
SEQ = 200
SEQ_PAD = 208
EMBED_DIM = 32
NUM_CLASSES = 100
CLASS_PAD = 128

NUM_CORES = 2
NUM_SUBCORES = 16
NUM_WORKERS = NUM_CORES * NUM_SUBCORES
BPW = BATCH // NUM_WORKERS

_N16 = SEQ_PAD // 16
_G1 = 128
_G2 = SEQ_PAD - _G1
_NBUF = 8


def _sc_pool_sums(x_pad, table):
    mesh = plsc.VectorSubcoreMesh(core_axis_name="c", subcore_axis_name="s")

    @functools.partial(
        pl.kernel,
        mesh=mesh,
        compiler_params=pltpu.CompilerParams(use_tc_tiling_on_sc=False),
        out_type=jax.ShapeDtypeStruct((BATCH, EMBED_DIM), jnp.float32),
        scratch_types=[
            pltpu.VMEM((BPW, SEQ_PAD), jnp.int32),
            pltpu.VMEM((SEQ_PAD, EMBED_DIM), jnp.float32),
            pltpu.VMEM((BPW, EMBED_DIM), jnp.float32),
            pltpu.SemaphoreType.DMA,
        ],
    )
    def body(x_hbm, table_hbm, out_hbm, idx_v, rows_v, sums_v, sem):
        wid = lax.axis_index("s") * NUM_CORES + lax.axis_index("c")
        base = wid * BPW
        pltpu.sync_copy(x_hbm.at[pl.ds(base, BPW)], idx_v)

        zero = jnp.zeros((16,), jnp.float32)

        def row_body(row, carry):
            cp1 = pltpu.async_copy(
                table_hbm.at[idx_v.at[row, pl.ds(0, _G1)]],
                rows_v.at[pl.ds(0, _G1)], sem)
            cp2 = pltpu.async_copy(
                table_hbm.at[idx_v.at[row, pl.ds(_G1, _G2)]],
                rows_v.at[pl.ds(_G1, _G2)], sem)
            cp1.wait()
            cp2.wait()

            def chunk(c, accs):
                a0, a1, a2, a3, a4, a5, a6, a7 = accs
                r0 = c * 16
                for u in range(0, 16, 4):
                    a0 = a0 + rows_v[r0 + u, pl.ds(0, 16)]
                    a1 = a1 + rows_v[r0 + u, pl.ds(16, 16)]
                    a2 = a2 + rows_v[r0 + u + 1, pl.ds(0, 16)]
                    a3 = a3 + rows_v[r0 + u + 1, pl.ds(16, 16)]
                    a4 = a4 + rows_v[r0 + u + 2, pl.ds(0, 16)]
                    a5 = a5 + rows_v[r0 + u + 2, pl.ds(16, 16)]
                    a6 = a6 + rows_v[r0 + u + 3, pl.ds(0, 16)]
                    a7 = a7 + rows_v[r0 + u + 3, pl.ds(16, 16)]
                return (a0, a1, a2, a3, a4, a5, a6, a7)

            accs = lax.fori_loop(0, _N16, chunk, (zero,) * 8)

            s0 = (accs[0] + accs[2]) + (accs[4] + accs[6])
            s1 = (accs[1] + accs[3]) + (accs[5] + accs[7])
            sums_v[row, pl.ds(0, 16)] = s0
            sums_v[row, pl.ds(16, 16)] = s1
            return carry

        lax.fori_loop(0, BPW, row_body, 0)
        pltpu.sync_copy(sums_v, out_hbm.at[pl.ds(base, BPW)])

    return body(x_pad, table)


def _tc_matmul(sums, x_pad, t0, w_scaled, b_pad):
    def body(s_ref, x_ref, t0_ref, w_ref, b_ref, o_ref):
        cnt0 = jnp.sum((x_ref[...] == 0).astype(jnp.float32), axis=1,
                       keepdims=True)
        pooled = s_ref[...] - cnt0 * t0_ref[...]
        o_ref[...] = jnp.dot(
            pooled, w_ref[...], preferred_element_type=jnp.float32
        ) + b_ref[...]

    blk = 1024
    return pl.pallas_call(
        body,
        grid=(BATCH // blk,),
        in_specs=[
            pl.BlockSpec((blk, EMBED_DIM), lambda i: (i, 0)),
            pl.BlockSpec((blk, SEQ_PAD), lambda i: (i, 0)),
            pl.BlockSpec((1, EMBED_DIM), lambda i: (0, 0)),
            pl.BlockSpec((EMBED_DIM, CLASS_PAD), lambda i: (0, 0)),
            pl.BlockSpec((1, CLASS_PAD), lambda i: (0, 0)),
        ],
        out_specs=pl.BlockSpec((blk, CLASS_PAD), lambda i: (i, 0)),
        out_shape=jax.ShapeDtypeStruct((BATCH, CLASS_PAD), jnp.float32),
    )(sums, x_pad, t0, w_scaled, b_pad)


def kernel(x, table, W, b):
    x_pad = jnp.pad(x, ((0, 0), (0, SEQ_PAD - SEQ)))
    t0 = lax.slice(table, (0, 0), (1, EMBED_DIM))
    w_scaled = jnp.zeros((EMBED_DIM, CLASS_PAD), jnp.float32)
    w_scaled = w_scaled.at[:, :NUM_CLASSES].set(W.T * (1.0 / SEQ))
    b_pad = jnp.zeros((1, CLASS_PAD), jnp.float32).at[0, :NUM_CLASSES].set(b)

    sums = _sc_pool_sums(x_pad, table)
    logits_pad = _tc_matmul(sums, x_pad, t0, w_scaled, b_pad)
    return logits_pad[:, :NUM_CLASSES]


# speedup vs baseline: 12.2313x; 1.3360x over previous
import functools

import jax
import jax.numpy as jnp
from jax import lax
from jax.experimental import pallas as pl
from jax.experimental.pallas import tpu as pltpu
from jax.experimental.pallas import tpu_sc as plsc

BATCH = 4096
SEQ = 200
EMBED_DIM = 32
NUM_CLASSES = 100
CLASS_PAD = 128

NUM_CORES = 2
NUM_SUBCORES = 16
NUM_WORKERS = NUM_CORES * NUM_SUBCORES
BPW = BATCH // NUM_WORKERS

_N16 = SEQ // 16
_TAIL = SEQ - _N16 * 16
_G1 = 128
_G2 = SEQ - _G1


def _sc_pool_sums(x_pad, table):
    mesh = plsc.VectorSubcoreMesh(core_axis_name="c", subcore_axis_name="s")

    @functools.partial(
        pl.kernel,
        mesh=mesh,
        compiler_params=pltpu.CompilerParams(use_tc_tiling_on_sc=False),
        out_type=jax.ShapeDtypeStruct((BATCH, EMBED_DIM), jnp.float32),
        scratch_types=[
            pltpu.VMEM((BPW, SEQ), jnp.int32),
            pltpu.VMEM((SEQ, EMBED_DIM), jnp.float32),
            pltpu.VMEM((BPW, EMBED_DIM), jnp.float32),
            pltpu.SemaphoreType.DMA,
        ],
    )
    def body(x_hbm, table_hbm, out_hbm, idx_v, rows_v, sums_v, sem):
        wid = lax.axis_index("s") * NUM_CORES + lax.axis_index("c")
        base = wid * BPW
        pltpu.sync_copy(x_hbm.at[pl.ds(base, BPW)], idx_v)

        zero = jnp.zeros((16,), jnp.float32)

        def row_body(row, carry):
            cp1 = pltpu.async_copy(
                table_hbm.at[idx_v.at[row, pl.ds(0, _G1)]],
                rows_v.at[pl.ds(0, _G1)], sem)
            cp2 = pltpu.async_copy(
                table_hbm.at[idx_v.at[row, pl.ds(_G1, _G2)]],
                rows_v.at[pl.ds(_G1, _G2)], sem)
            cp1.wait()
            cp2.wait()

            def chunk(c, accs):
                a0, a1, a2, a3, a4, a5, a6, a7 = accs
                r0 = c * 16
                for u in range(0, 16, 4):
                    a0 = a0 + rows_v[r0 + u, pl.ds(0, 16)]
                    a1 = a1 + rows_v[r0 + u, pl.ds(16, 16)]
                    a2 = a2 + rows_v[r0 + u + 1, pl.ds(0, 16)]
                    a3 = a3 + rows_v[r0 + u + 1, pl.ds(16, 16)]
                    a4 = a4 + rows_v[r0 + u + 2, pl.ds(0, 16)]
                    a5 = a5 + rows_v[r0 + u + 2, pl.ds(16, 16)]
                    a6 = a6 + rows_v[r0 + u + 3, pl.ds(0, 16)]
                    a7 = a7 + rows_v[r0 + u + 3, pl.ds(16, 16)]
                return (a0, a1, a2, a3, a4, a5, a6, a7)

            accs = lax.fori_loop(0, _N16, chunk, (zero,) * 8)

            accs = list(accs)
            for u in range(_TAIL):
                p = (u % 4) * 2
                accs[p] = accs[p] + rows_v[_N16 * 16 + u, pl.ds(0, 16)]
                accs[p + 1] = accs[p + 1] + rows_v[
                    _N16 * 16 + u, pl.ds(16, 16)]

            s0 = (accs[0] + accs[2]) + (accs[4] + accs[6])
            s1 = (accs[1] + accs[3]) + (accs[5] + accs[7])
            sums_v[row, pl.ds(0, 16)] = s0
            sums_v[row, pl.ds(16, 16)] = s1
            return carry

        lax.fori_loop(0, BPW, row_body, 0)
        pltpu.sync_copy(sums_v, out_hbm.at[pl.ds(base, BPW)])

    return body(x_pad, table)


def _tc_matmul(sums, x, t0, w_scaled, b_pad):
    def body(s_ref, x_ref, t0_ref, w_ref, b_ref, o_ref):
        cnt0 = jnp.sum((x_ref[...] == 0).astype(jnp.float32), axis=1,
                       keepdims=True)
        pooled = s_ref[...] - cnt0 * t0_ref[...]
        o_ref[...] = jnp.dot(
            pooled, w_ref[...], preferred_element_type=jnp.float32
        ) + b_ref[...]

    blk = 1024
    return pl.pallas_call(
        body,
        grid=(BATCH // blk,),
        in_specs=[
            pl.BlockSpec((blk, EMBED_DIM), lambda i: (i, 0)),
            pl.BlockSpec((blk, SEQ), lambda i: (i, 0)),
            pl.BlockSpec((1, EMBED_DIM), lambda i: (0, 0)),
            pl.BlockSpec((EMBED_DIM, CLASS_PAD), lambda i: (0, 0)),
            pl.BlockSpec((1, CLASS_PAD), lambda i: (0, 0)),
        ],
        out_specs=pl.BlockSpec((blk, CLASS_PAD), lambda i: (i, 0)),
        out_shape=jax.ShapeDtypeStruct((BATCH, CLASS_PAD), jnp.float32),
    )(sums, x, t0, w_scaled, b_pad)


def kernel(x, table, W, b):
    t0 = lax.slice(table, (0, 0), (1, EMBED_DIM))
    w_scaled = jnp.zeros((EMBED_DIM, CLASS_PAD), jnp.float32)
    w_scaled = w_scaled.at[:, :NUM_CLASSES].set(W.T * (1.0 / SEQ))
    b_pad = jnp.zeros((1, CLASS_PAD), jnp.float32).at[0, :NUM_CLASSES].set(b)

    sums = _sc_pool_sums(x, table)
    logits_pad = _tc_matmul(sums, x, t0, w_scaled, b_pad)
    return logits_pad[:, :NUM_CLASSES]
